# baseline (speedup 1.0000x reference)
import functools

import jax
import jax.numpy as jnp
import numpy as np
from jax import lax
from jax.experimental import pallas as pl
from jax.experimental.pallas import tpu as pltpu
from jax.experimental.pallas import tpu_sc as plsc

NC, NS, L = 2, 16, 16
NW = NC * NS
B, H, W, P = 16, 512, 512, 100000
HALF = P // 2
HW = H * W
PACKED_PER_MAP = HW // 2
SC_WORDS = 8 * PACKED_PER_MAP
STAGE = SC_WORDS // NS
CHUNK = 4000
CHUNK_SIZES = [CHUNK] * 12 + [HALF - 12 * CHUNK]


def _softplus(s):
    v = jnp.exp(-jnp.abs(s))
    r = v / (v + 2.0)
    r2 = r * r
    poly = 1.0 + r2 * (1.0 / 3.0 + r2 * (1.0 / 5.0 + r2 * (1.0 / 7.0 + r2 * (1.0 / 9.0))))
    return jnp.maximum(s, 0.0) + 2.0 * r * poly


def _unpack_half(w, sel):
    sh = (sel ^ 1) << 4
    return lax.bitcast_convert_type((w << sh) & jnp.int32(-65536), jnp.float32)


def _sc_body(packed_hbm, rel_hbm, xa_hbm, ya_hbm, xb_hbm, yb_hbm, out_hbm,
             shared, bufx, bufy, idxa, idxb, bufr, bufsel, zwa, zwb, accv, sems):
    c = lax.axis_index("c")
    s = lax.axis_index("s")
    wid = s * NC + c
    b = c * 8 + (s // 2)
    base = b * P + (s % 2) * HALF
    gbase = (s // 2) * PACKED_PER_MAP

    hoff = pl.multiple_of(c * SC_WORDS + s * STAGE, 8)
    soff = pl.multiple_of(s * STAGE, 8)
    pltpu.sync_copy(packed_hbm.at[pl.ds(hoff, STAGE)], shared.at[pl.ds(soff, STAGE)])
    plsc.subcore_barrier()

    def stage_and_build(k, ring):
        n = CHUNK_SIZES[k]
        nvec = n // L
        off = pl.multiple_of(base + k * CHUNK, 8)
        pltpu.sync_copy(xa_hbm.at[pl.ds(off, n)], bufx.at[pl.ds(0, n)])
        pltpu.sync_copy(ya_hbm.at[pl.ds(off, n)], bufy.at[pl.ds(0, n)])

        def mk_a(i, _):
            sl = pl.ds(pl.multiple_of(i * L, L), L)
            lin = bufx[sl] * W + bufy[sl]
            idxa[ring][sl] = gbase + (lin >> 1)
            bufsel[ring][sl] = lin & 1
            return 0

        lax.fori_loop(0, nvec, mk_a, 0)
        pltpu.sync_copy(xb_hbm.at[pl.ds(off, n)], bufx.at[pl.ds(0, n)])
        pltpu.sync_copy(yb_hbm.at[pl.ds(off, n)], bufy.at[pl.ds(0, n)])

        def mk_b(i, _):
            sl = pl.ds(pl.multiple_of(i * L, L), L)
            lin = bufx[sl] * W + bufy[sl]
            idxb[ring][sl] = gbase + (lin >> 1)
            bufsel[ring][sl] = bufsel[ring][sl] | ((lin & 1) << 1)
            return 0

        lax.fori_loop(0, nvec, mk_b, 0)
        pltpu.sync_copy(rel_hbm.at[pl.ds(off, n)], bufr[ring].at[pl.ds(0, n)])

    def fire(ring):
        return (pltpu.async_copy(shared.at[idxa[ring]], zwa[ring], sems[2 * ring]),
                pltpu.async_copy(shared.at[idxb[ring]], zwb[ring], sems[2 * ring + 1]))

    def accumulate(k, ring, carry):
        def acc_step(i, cr):
            a_log, a_nnz, a_sq, a_nze = cr
            sl = pl.ds(pl.multiple_of(i * L, L), L)
            r = bufr[ring][sl]
            selw = bufsel[ring][sl]
            va = _unpack_half(zwa[ring][sl], selw & 1)
            vb = _unpack_half(zwb[ring][sl], (selw >> 1) & 1)
            pred = va - vb
            t = r.astype(jnp.float32)
            nz = (r == 1) | (r == -1)
            ze = r == 0
            sp = _softplus(-t * pred)
            one = jnp.ones((L,), jnp.float32)
            a_log = a_log + jnp.where(nz, sp, 0.0)
            a_nnz = a_nnz + jnp.where(nz, one, 0.0)
            a_sq = a_sq + jnp.where(ze, pred * pred, 0.0)
            a_nze = a_nze + jnp.where(ze, one, 0.0)
            return a_log, a_nnz, a_sq, a_nze

        return lax.fori_loop(0, CHUNK_SIZES[k] // L, acc_step, carry)

    zero = jnp.zeros((L,), jnp.float32)
    carry = (zero, zero, zero, zero)

    stage_and_build(0, 0)
    inflight = fire(0)
    nchunk = len(CHUNK_SIZES)
    for k in range(nchunk):
        ring, nring = k % 2, (k + 1) % 2
        if k + 1 < nchunk:
            stage_and_build(k + 1, nring)
        for cp in inflight:
            cp.wait()
        if k + 1 < nchunk:
            inflight = fire(nring)
        carry = accumulate(k, ring, carry)

    acc_log, acc_nnz, acc_sq, acc_nze = carry
    accv[0, :] = acc_log
    accv[1, :] = acc_nnz
    accv[2, :] = acc_sq
    accv[3, :] = acc_nze
    pltpu.sync_copy(accv, out_hbm.at[wid])


@functools.partial(jax.jit, static_argnames=())
def kernel(output, ordinal_relation, x_A, y_A, x_B, y_B):
    packed = lax.bitcast_convert_type(
        output.astype(jnp.bfloat16).reshape(B * HW // 2, 2), jnp.int32)
    rel = ordinal_relation.reshape(B * P)
    xa = x_A.reshape(B * P)
    ya = y_A.reshape(B * P)
    xb = x_B.reshape(B * P)
    yb = y_B.reshape(B * P)

    sc = pl.kernel(
        _sc_body,
        out_type=jax.ShapeDtypeStruct((NW, 4, L), jnp.float32),
        mesh=plsc.VectorSubcoreMesh(core_axis_name="c", subcore_axis_name="s"),
        scratch_types=[
            pltpu.VMEM_SHARED((SC_WORDS,), jnp.int32),
            pltpu.VMEM((CHUNK,), jnp.int32),
            pltpu.VMEM((CHUNK,), jnp.int32),
            [pltpu.VMEM((CHUNK,), jnp.int32)] * 2,
            [pltpu.VMEM((CHUNK,), jnp.int32)] * 2,
            [pltpu.VMEM((CHUNK,), jnp.int32)] * 2,
            [pltpu.VMEM((CHUNK,), jnp.int32)] * 2,
            [pltpu.VMEM((CHUNK,), jnp.int32)] * 2,
            [pltpu.VMEM((CHUNK,), jnp.int32)] * 2,
            pltpu.VMEM((4, L), jnp.float32),
            [pltpu.SemaphoreType.DMA] * 4,
        ],
    )
    acc = sc(packed, rel, xa, ya, xb, yb)
    per_w = acc.sum(axis=-1)
    b_of_wid = np.array([(w % NC) * 8 + (w // NC) // 2 for w in range(NW)])
    part = jnp.zeros((B, 4), jnp.float32).at[b_of_wid].add(per_w)
    loss = part[:, 0] / part[:, 1] + part[:, 2] / part[:, 3]
    return jnp.sum(loss) / jnp.float32(B)


# speedup vs baseline: 7.6501x; 7.6501x over previous
import functools

import jax
import jax.numpy as jnp
from jax import lax
from jax.experimental import pallas as pl
from jax.experimental.pallas import tpu as pltpu
from jax.experimental.pallas import tpu_sc as plsc

NC, NS, L = 2, 16, 16
NW = NC * NS
B, H, W, P = 16, 512, 512, 100000
HALF = P // 2
CMAX = 10000
CHUNK_SIZES = [2000, 4000, 8000, 10000, 10000, 10000, 6000]
CHUNK_OFFS = [sum(CHUNK_SIZES[:i]) for i in range(len(CHUNK_SIZES))]


def _softplus(s):
    v = jnp.exp(-jnp.abs(s))
    r = v / (v + 2.0)
    r2 = r * r
    poly = 1.0 + r2 * (1.0 / 3.0 + r2 * (1.0 / 5.0 + r2 * (1.0 / 7.0 + r2 * (1.0 / 9.0))))
    return jnp.maximum(s, 0.0) + 2.0 * r * poly


def _sc_body(depth_hbm, rel_hbm, xa_hbm, ya_hbm, xb_hbm, yb_hbm, out_hbm,
             bufx, bufy, idxa, idxb, bufr, za, zb, accv, sems):
    wid = lax.axis_index("s") * NC + lax.axis_index("c")
    b = wid // 2
    base = b * P + (wid % 2) * HALF
    gbase = b * (H * W)

    def stage_build_fire(k, ring):
        n = CHUNK_SIZES[k]
        nvec = n // L
        off = pl.multiple_of(base + CHUNK_OFFS[k], 8)

        def build(xh, yh, dst):
            pltpu.sync_copy(xh.at[pl.ds(off, n)], bufx.at[pl.ds(0, n)])
            pltpu.sync_copy(yh.at[pl.ds(off, n)], bufy.at[pl.ds(0, n)])

            def mk(i, _):
                sl = pl.ds(pl.multiple_of(i * L, L), L)
                dst[sl] = gbase + bufx[sl] * W + bufy[sl]
                return 0

            lax.fori_loop(0, nvec, mk, 0)

        build(xa_hbm, ya_hbm, idxa[ring])
        cp_a = pltpu.async_copy(
            depth_hbm.at[idxa[ring].at[pl.ds(0, n)]], za[ring].at[pl.ds(0, n)],
            sems[2 * ring])
        build(xb_hbm, yb_hbm, idxb[ring])
        cp_b = pltpu.async_copy(
            depth_hbm.at[idxb[ring].at[pl.ds(0, n)]], zb[ring].at[pl.ds(0, n)],
            sems[2 * ring + 1])
        pltpu.sync_copy(rel_hbm.at[pl.ds(off, n)], bufr[ring].at[pl.ds(0, n)])
        return (cp_a, cp_b)

    def accumulate(k, ring, carry):
        def acc_step(i, cr):
            a_log, a_nnz, a_sq, a_nze = cr
            sl = pl.ds(pl.multiple_of(i * L, L), L)
            r = bufr[ring][sl]
            pred = za[ring][sl] - zb[ring][sl]
            t = r.astype(jnp.float32)
            nz = (r == 1) | (r == -1)
            ze = r == 0
            sp = _softplus(-t * pred)
            one = jnp.ones((L,), jnp.float32)
            a_log = a_log + jnp.where(nz, sp, 0.0)
            a_nnz = a_nnz + jnp.where(nz, one, 0.0)
            a_sq = a_sq + jnp.where(ze, pred * pred, 0.0)
            a_nze = a_nze + jnp.where(ze, one, 0.0)
            return a_log, a_nnz, a_sq, a_nze

        return lax.fori_loop(0, CHUNK_SIZES[k] // L, acc_step, carry)

    zero = jnp.zeros((L,), jnp.float32)
    carry = (zero, zero, zero, zero)

    inflight = stage_build_fire(0, 0)
    nchunk = len(CHUNK_SIZES)
    for k in range(nchunk):
        ring, nring = k % 2, (k + 1) % 2
        nxt = stage_build_fire(k + 1, nring) if k + 1 < nchunk else None
        for cp in inflight:
            cp.wait()
        carry = accumulate(k, ring, carry)
        inflight = nxt

    acc_log, acc_nnz, acc_sq, acc_nze = carry
    accv[0, :] = acc_log
    accv[1, :] = acc_nnz
    accv[2, :] = acc_sq
    accv[3, :] = acc_nze
    pltpu.sync_copy(accv, out_hbm.at[wid])


@functools.partial(jax.jit, static_argnames=())
def kernel(output, ordinal_relation, x_A, y_A, x_B, y_B):
    depth = output.reshape(B * H * W)
    rel = ordinal_relation.reshape(B * P)
    xa = x_A.reshape(B * P)
    ya = y_A.reshape(B * P)
    xb = x_B.reshape(B * P)
    yb = y_B.reshape(B * P)

    sc = pl.kernel(
        _sc_body,
        out_type=jax.ShapeDtypeStruct((NW, 4, L), jnp.float32),
        mesh=plsc.VectorSubcoreMesh(core_axis_name="c", subcore_axis_name="s"),
        scratch_types=[
            pltpu.VMEM((CMAX,), jnp.int32),
            pltpu.VMEM((CMAX,), jnp.int32),
            [pltpu.VMEM((CMAX,), jnp.int32)] * 2,
            [pltpu.VMEM((CMAX,), jnp.int32)] * 2,
            [pltpu.VMEM((CMAX,), jnp.int32)] * 2,
            [pltpu.VMEM((CMAX,), jnp.float32)] * 2,
            [pltpu.VMEM((CMAX,), jnp.float32)] * 2,
            pltpu.VMEM((4, L), jnp.float32),
            [pltpu.SemaphoreType.DMA] * 4,
        ],
    )
    acc = sc(depth, rel, xa, ya, xb, yb)
    part = acc.sum(axis=-1).reshape(B, 2, 4).sum(axis=1)
    loss = part[:, 0] / part[:, 1] + part[:, 2] / part[:, 3]
    return jnp.sum(loss) / jnp.float32(B)


# speedup vs baseline: 7.6501x; 1.0000x over previous
import functools

import jax
import jax.numpy as jnp
from jax import lax
from jax.experimental import pallas as pl
from jax.experimental.pallas import tpu as pltpu
from jax.experimental.pallas import tpu_sc as plsc

NC, NS, L = 2, 16, 16
NW = NC * NS
B, H, W, P = 16, 512, 512, 100000
HALF = P // 2
CMAX = 7504
CHUNK_SIZES = [2000, 4992, 7504, 7504, 7504, 7504, 7504, 5488]
CHUNK_OFFS = [sum(CHUNK_SIZES[:i]) for i in range(len(CHUNK_SIZES))]
NRING = 3


def _softplus(s):
    v = jnp.exp(-jnp.abs(s))
    r = v / (v + 2.0)
    r2 = r * r
    poly = 1.0 + r2 * (1.0 / 3.0 + r2 * (1.0 / 5.0 + r2 * (1.0 / 7.0 + r2 * (1.0 / 9.0))))
    return jnp.maximum(s, 0.0) + 2.0 * r * poly


def _sc_body(depth_hbm, rel_hbm, xa_hbm, ya_hbm, xb_hbm, yb_hbm, out_hbm,
             bufx, bufy, idxa, idxb, bufr, za, zb, accv, sems):
    wid = lax.axis_index("s") * NC + lax.axis_index("c")
    b = wid // 2
    base = b * P + (wid % 2) * HALF
    gbase = b * (H * W)

    def stage_build_fire(k, ring):
        n = CHUNK_SIZES[k]
        nvec = n // L
        off = pl.multiple_of(base + CHUNK_OFFS[k], 8)

        def build(xh, yh, dst):
            pltpu.sync_copy(xh.at[pl.ds(off, n)], bufx.at[pl.ds(0, n)])
            pltpu.sync_copy(yh.at[pl.ds(off, n)], bufy.at[pl.ds(0, n)])

            def mk(i, _):
                sl = pl.ds(pl.multiple_of(i * L, L), L)
                dst[sl] = gbase + bufx[sl] * W + bufy[sl]
                return 0

            lax.fori_loop(0, nvec, mk, 0)

        build(xa_hbm, ya_hbm, idxa[ring])
        cp_a = pltpu.async_copy(
            depth_hbm.at[idxa[ring].at[pl.ds(0, n)]], za[ring].at[pl.ds(0, n)],
            sems[2 * ring])
        build(xb_hbm, yb_hbm, idxb[ring])
        cp_b = pltpu.async_copy(
            depth_hbm.at[idxb[ring].at[pl.ds(0, n)]], zb[ring].at[pl.ds(0, n)],
            sems[2 * ring + 1])
        pltpu.sync_copy(rel_hbm.at[pl.ds(off, n)], bufr[ring].at[pl.ds(0, n)])
        return (cp_a, cp_b)

    def accumulate(k, ring, carry):
        def acc_step(i, cr):
            a_log, a_nnz, a_sq, a_nze = cr
            sl = pl.ds(pl.multiple_of(i * L, L), L)
            r = bufr[ring][sl]
            pred = za[ring][sl] - zb[ring][sl]
            t = r.astype(jnp.float32)
            nz = (r == 1) | (r == -1)
            ze = r == 0
            sp = _softplus(-t * pred)
            one = jnp.ones((L,), jnp.float32)
            a_log = a_log + jnp.where(nz, sp, 0.0)
            a_nnz = a_nnz + jnp.where(nz, one, 0.0)
            a_sq = a_sq + jnp.where(ze, pred * pred, 0.0)
            a_nze = a_nze + jnp.where(ze, one, 0.0)
            return a_log, a_nnz, a_sq, a_nze

        return lax.fori_loop(0, CHUNK_SIZES[k] // L, acc_step, carry)

    zero = jnp.zeros((L,), jnp.float32)
    carry = (zero, zero, zero, zero)

    nchunk = len(CHUNK_SIZES)
    queue = [stage_build_fire(0, 0), stage_build_fire(1, 1)]
    for k in range(nchunk):
        if k + 2 < nchunk:
            queue.append(stage_build_fire(k + 2, (k + 2) % NRING))
        for cp in queue.pop(0):
            cp.wait()
        carry = accumulate(k, k % NRING, carry)

    acc_log, acc_nnz, acc_sq, acc_nze = carry
    accv[0, :] = acc_log
    accv[1, :] = acc_nnz
    accv[2, :] = acc_sq
    accv[3, :] = acc_nze
    pltpu.sync_copy(accv, out_hbm.at[wid])


@functools.partial(jax.jit, static_argnames=())
def kernel(output, ordinal_relation, x_A, y_A, x_B, y_B):
    depth = output.reshape(B * H * W)
    rel = ordinal_relation.reshape(B * P)
    xa = x_A.reshape(B * P)
    ya = y_A.reshape(B * P)
    xb = x_B.reshape(B * P)
    yb = y_B.reshape(B * P)

    sc = pl.kernel(
        _sc_body,
        out_type=jax.ShapeDtypeStruct((NW, 4, L), jnp.float32),
        mesh=plsc.VectorSubcoreMesh(core_axis_name="c", subcore_axis_name="s"),
        scratch_types=[
            pltpu.VMEM((CMAX,), jnp.int32),
            pltpu.VMEM((CMAX,), jnp.int32),
            [pltpu.VMEM((CMAX,), jnp.int32)] * NRING,
            [pltpu.VMEM((CMAX,), jnp.int32)] * NRING,
            [pltpu.VMEM((CMAX,), jnp.int32)] * NRING,
            [pltpu.VMEM((CMAX,), jnp.float32)] * NRING,
            [pltpu.VMEM((CMAX,), jnp.float32)] * NRING,
            pltpu.VMEM((4, L), jnp.float32),
            [pltpu.SemaphoreType.DMA] * (2 * NRING),
        ],
    )
    acc = sc(depth, rel, xa, ya, xb, yb)
    part = acc.sum(axis=-1).reshape(B, 2, 4).sum(axis=1)
    loss = part[:, 0] / part[:, 1] + part[:, 2] / part[:, 3]
    return jnp.sum(loss) / jnp.float32(B)


# speedup vs baseline: 8.4024x; 1.0983x over previous
import functools

import jax
import jax.numpy as jnp
from jax import lax
from jax.experimental import pallas as pl
from jax.experimental.pallas import tpu as pltpu
from jax.experimental.pallas import tpu_sc as plsc

NC, NS, L = 2, 16, 16
NW = NC * NS
B, H, W, P = 16, 512, 512, 100000
HALF = P // 2
IDXBITS = 22
IDXMASK = (1 << IDXBITS) - 1
CMAX = 8336
CHUNK_SIZES = [2000, 4992, 8336, 8336, 8336, 8336, 8336, 1328]
CHUNK_OFFS = [sum(CHUNK_SIZES[:i]) for i in range(len(CHUNK_SIZES))]
NRING = 3


def _softplus(s):
    v = jnp.exp(-jnp.abs(s))
    r = v / (v + 2.0)
    r2 = r * r
    poly = 1.0 + r2 * (1.0 / 3.0 + r2 * (1.0 / 5.0 + r2 * (1.0 / 7.0 + r2 * (1.0 / 9.0))))
    return jnp.maximum(s, 0.0) + 2.0 * r * poly


def _sc_body(depth_hbm, wa_hbm, wb_hbm, out_hbm,
             bufwa, bufwb, idxa, za, zb, accv, sems):
    wid = lax.axis_index("s") * NC + lax.axis_index("c")
    base = (wid // 2) * P + (wid % 2) * HALF

    def stage_build_fire(k, ring):
        n = CHUNK_SIZES[k]
        off = pl.multiple_of(base + CHUNK_OFFS[k], 8)
        pltpu.sync_copy(wa_hbm.at[pl.ds(off, n)], bufwa[ring].at[pl.ds(0, n)])

        def mk(i, _):
            sl = pl.ds(pl.multiple_of(i * L, L), L)
            idxa[ring][sl] = bufwa[ring][sl] & IDXMASK
            return 0

        lax.fori_loop(0, n // L, mk, 0)
        cp_a = pltpu.async_copy(
            depth_hbm.at[idxa[ring].at[pl.ds(0, n)]], za[ring].at[pl.ds(0, n)],
            sems[2 * ring])
        pltpu.sync_copy(wb_hbm.at[pl.ds(off, n)], bufwb[ring].at[pl.ds(0, n)])
        cp_b = pltpu.async_copy(
            depth_hbm.at[bufwb[ring].at[pl.ds(0, n)]], zb[ring].at[pl.ds(0, n)],
            sems[2 * ring + 1])
        return (cp_a, cp_b)

    def accumulate(k, ring, carry):
        def acc_step(i, cr):
            a_log, a_nnz, a_sq, a_nze = cr
            sl = pl.ds(pl.multiple_of(i * L, L), L)
            r = lax.shift_right_logical(bufwa[ring][sl], IDXBITS)
            pred = za[ring][sl] - zb[ring][sl]
            t = (r - 1).astype(jnp.float32)
            nz = (r & 1) == 0
            ze = r == 1
            sp = _softplus(-t * pred)
            one = jnp.ones((L,), jnp.float32)
            a_log = a_log + jnp.where(nz, sp, 0.0)
            a_nnz = a_nnz + jnp.where(nz, one, 0.0)
            a_sq = a_sq + jnp.where(ze, pred * pred, 0.0)
            a_nze = a_nze + jnp.where(ze, one, 0.0)
            return a_log, a_nnz, a_sq, a_nze

        return lax.fori_loop(0, CHUNK_SIZES[k] // L, acc_step, carry)

    zero = jnp.zeros((L,), jnp.float32)
    carry = (zero, zero, zero, zero)

    nchunk = len(CHUNK_SIZES)
    queue = [stage_build_fire(0, 0), stage_build_fire(1, 1)]
    for k in range(nchunk):
        if k + 2 < nchunk:
            queue.append(stage_build_fire(k + 2, (k + 2) % NRING))
        for cp in queue.pop(0):
            cp.wait()
        carry = accumulate(k, k % NRING, carry)

    acc_log, acc_nnz, acc_sq, acc_nze = carry
    accv[0, :] = acc_log
    accv[1, :] = acc_nnz
    accv[2, :] = acc_sq
    accv[3, :] = acc_nze
    pltpu.sync_copy(accv, out_hbm.at[wid])


@functools.partial(jax.jit, static_argnames=())
def kernel(output, ordinal_relation, x_A, y_A, x_B, y_B):
    depth = output.reshape(B * H * W)
    gb = (jnp.arange(B, dtype=jnp.int32) * (H * W))[:, None]
    wa = ((gb + x_A * W + y_A)
          | ((ordinal_relation + 1) << IDXBITS)).reshape(B * P)
    wb = (gb + x_B * W + y_B).reshape(B * P)

    sc = pl.kernel(
        _sc_body,
        out_type=jax.ShapeDtypeStruct((NW, 4, L), jnp.float32),
        mesh=plsc.VectorSubcoreMesh(core_axis_name="c", subcore_axis_name="s"),
        scratch_types=[
            [pltpu.VMEM((CMAX,), jnp.int32)] * NRING,
            [pltpu.VMEM((CMAX,), jnp.int32)] * NRING,
            [pltpu.VMEM((CMAX,), jnp.int32)] * NRING,
            [pltpu.VMEM((CMAX,), jnp.float32)] * NRING,
            [pltpu.VMEM((CMAX,), jnp.float32)] * NRING,
            pltpu.VMEM((4, L), jnp.float32),
            [pltpu.SemaphoreType.DMA] * (2 * NRING),
        ],
    )
    acc = sc(depth, wa, wb)
    part = acc.sum(axis=-1).reshape(B, 2, 4).sum(axis=1)
    loss = part[:, 0] / part[:, 1] + part[:, 2] / part[:, 3]
    return jnp.sum(loss) / jnp.float32(B)
